# Initial kernel scaffold; baseline (speedup 1.0000x reference)
import jax
import jax.numpy as jnp
from jax.experimental import pallas as pl


def kernel(x, edge_index, batch, W1, b1, W2, b2, Wfc, bfc):
    raise NotImplementedError("write your pallas kernel here")


# speedup vs baseline: 2.2612x; 2.2612x over previous
import functools

import jax
import jax.numpy as jnp
from jax import lax
from jax.experimental import pallas as pl
from jax.experimental.pallas import tpu as pltpu
from jax.experimental.pallas import tpu_sc as plsc

N = 10000
E = 160000
D_IN = 256
D_H = 512
N_CLASSES = 64
N_GRAPHS = 64

NC = 2
NS = 16
LW = 128

E_PAD = 163840
CH = 128
PER_TILE = E_PAD // NS
N_CHUNKS = PER_TILE // CH
ACC_ROWS = 10112
DUMMY_DST = N + 48

BM = 1000
MB = N // BM


@functools.lru_cache(maxsize=None)
def _make_agg(n_slices):
  mesh = plsc.VectorSubcoreMesh(
      core_axis_name="c", subcore_axis_name="s", num_cores=NC, num_subcores=NS)

  @functools.partial(
      pl.kernel,
      out_type=jax.ShapeDtypeStruct((n_slices, N, LW), jnp.float32),
      mesh=mesh,
      scratch_types=[
          pltpu.VMEM((CH,), jnp.int32),
          pltpu.VMEM((CH,), jnp.int32),
          pltpu.VMEM((CH, LW), jnp.float32),
          pltpu.VMEM_SHARED((ACC_ROWS, LW), jnp.float32),
          pltpu.SemaphoreType.DMA,
      ],
  )
  def agg_kernel(table_hbm, src_hbm, dst_hbm, zeros_hbm, out_hbm,
                 sidx, didx, rows, acc, sem):
    c = lax.axis_index("c")
    s = lax.axis_index("s")
    zrows = ACC_ROWS // NS

    for i in range(n_slices // NC):
      sl = i * NC + c

      pltpu.sync_copy(zeros_hbm.at[pl.ds(s * zrows, zrows)],
                      acc.at[pl.ds(s * zrows, zrows)])
      plsc.subcore_barrier()

      def body(k, _):
        base = s * PER_TILE + k * CH
        pltpu.sync_copy(src_hbm.at[pl.ds(base, CH)], sidx)
        pltpu.async_copy(table_hbm.at[sl].at[sidx], rows, sem).wait()
        pltpu.sync_copy(dst_hbm.at[pl.ds(base, CH)], didx)
        pltpu.sync_copy(rows, acc.at[didx], add=True)
        return _

      lax.fori_loop(0, N_CHUNKS, body, 0)
      plsc.subcore_barrier()

      @pl.when(s == 0)
      def _():
        pltpu.sync_copy(acc.at[pl.ds(0, 640)], out_hbm.at[sl].at[pl.ds(0, 640)])

      @pl.when(s != 0)
      def _():
        base = 640 + (s - 1) * 624
        pltpu.sync_copy(acc.at[pl.ds(base, 624)],
                        out_hbm.at[sl].at[pl.ds(base, 624)])

      plsc.subcore_barrier()

  return agg_kernel


def _mm1_body(x_ref, agg_ref, w_ref, b_ref, out_ref):
  xa = x_ref[...] + jnp.concatenate([agg_ref[0], agg_ref[1]], axis=1)
  y = jnp.dot(xa, w_ref[...], preferred_element_type=jnp.float32)
  y = y + b_ref[0]
  out_ref[...] = jnp.maximum(y, 0.0)[None]


def _mm1(x, agg1, W1, b1r):
  return pl.pallas_call(
      _mm1_body,
      grid=(MB, D_H // LW),
      in_specs=[
          pl.BlockSpec((BM, D_IN), lambda m, j: (m, 0)),
          pl.BlockSpec((2, BM, LW), lambda m, j: (0, m, 0)),
          pl.BlockSpec((D_IN, LW), lambda m, j: (0, j)),
          pl.BlockSpec((1, 1, LW), lambda m, j: (j, 0, 0)),
      ],
      out_specs=pl.BlockSpec((1, BM, LW), lambda m, j: (j, m, 0)),
      out_shape=jax.ShapeDtypeStruct((D_H // LW, N, LW), jnp.float32),
  )(x, agg1, W1, b1r)


def _mm2_body(h1_ref, agg_ref, w_ref, b_ref, batch_ref, wfc_ref, bfc_ref,
              out_ref, s_acc, c_acc):
  m = pl.program_id(0)

  @pl.when(m == 0)
  def _():
    s_acc[...] = jnp.zeros_like(s_acc)
    c_acc[...] = jnp.zeros_like(c_acc)

  hk = jnp.concatenate([h1_ref[0], h1_ref[1], h1_ref[2], h1_ref[3]], axis=1)
  ak = jnp.concatenate([agg_ref[0], agg_ref[1], agg_ref[2], agg_ref[3]],
                       axis=1)
  y = jnp.dot(hk + ak, w_ref[...], preferred_element_type=jnp.float32)
  h2 = jnp.maximum(y + b_ref[...], 0.0)

  b = batch_ref[0, 0, :]
  gids = lax.broadcasted_iota(jnp.int32, (N_GRAPHS, BM), 0)
  oh = jnp.where(b[None, :] == gids, 1.0, 0.0).astype(jnp.float32)
  s_acc[...] += jnp.dot(oh, h2, preferred_element_type=jnp.float32)
  c_acc[...] += jnp.dot(oh, jnp.ones((BM, LW), jnp.float32),
                        preferred_element_type=jnp.float32)

  @pl.when(m == MB - 1)
  def _():
    cnt = jnp.maximum(c_acc[:, 0:1], 1.0)
    pooled = s_acc[...] / cnt
    logits = jnp.dot(pooled, wfc_ref[...], preferred_element_type=jnp.float32)
    logits = logits + bfc_ref[...]
    mx = jnp.max(logits, axis=1, keepdims=True)
    z = logits - mx
    lse = jnp.log(jnp.sum(jnp.exp(z), axis=1, keepdims=True))
    out_ref[...] = z - lse


def _mm2(h1, agg2, W2, b2r, batch3, Wfc, bfcr):
  return pl.pallas_call(
      _mm2_body,
      grid=(MB,),
      in_specs=[
          pl.BlockSpec((4, BM, LW), lambda m: (0, m, 0)),
          pl.BlockSpec((4, BM, LW), lambda m: (0, m, 0)),
          pl.BlockSpec((D_H, D_H), lambda m: (0, 0)),
          pl.BlockSpec((1, D_H), lambda m: (0, 0)),
          pl.BlockSpec((1, 1, BM), lambda m: (m, 0, 0)),
          pl.BlockSpec((D_H, N_CLASSES), lambda m: (0, 0)),
          pl.BlockSpec((1, N_CLASSES), lambda m: (0, 0)),
      ],
      out_specs=pl.BlockSpec((N_GRAPHS, N_CLASSES), lambda m: (0, 0)),
      out_shape=jax.ShapeDtypeStruct((N_GRAPHS, N_CLASSES), jnp.float32),
      scratch_shapes=[
          pltpu.VMEM((N_GRAPHS, D_H), jnp.float32),
          pltpu.VMEM((N_GRAPHS, LW), jnp.float32),
      ],
  )(h1, agg2, W2, b2r, batch3, Wfc, bfcr)


def kernel(x, edge_index, batch, W1, b1, W2, b2, Wfc, bfc):
  src = edge_index[0]
  dst = edge_index[1]
  pad = E_PAD - E
  src_p = jnp.concatenate([src, jnp.zeros((pad,), jnp.int32)])
  dst_p = jnp.concatenate([dst, jnp.full((pad,), DUMMY_DST, jnp.int32)])
  zeros_acc = jnp.zeros((ACC_ROWS, LW), jnp.float32)

  x_sl = x.reshape(N, D_IN // LW, LW).transpose(1, 0, 2)
  agg1 = _make_agg(2)(x_sl, src_p, dst_p, zeros_acc)
  h1 = _mm1(x, agg1, W1, b1.reshape(D_H // LW, 1, LW))
  agg2 = _make_agg(4)(h1, src_p, dst_p, zeros_acc)
  return _mm2(h1, agg2, W2, b2.reshape(1, D_H), batch.reshape(MB, 1, BM),
              Wfc, bfc.reshape(1, N_CLASSES))


# speedup vs baseline: 2.6830x; 1.1865x over previous
import functools

import jax
import jax.numpy as jnp
from jax import lax
from jax.experimental import pallas as pl
from jax.experimental.pallas import tpu as pltpu
from jax.experimental.pallas import tpu_sc as plsc

N = 10000
E = 160000
D_IN = 256
D_H = 512
N_CLASSES = 64
N_GRAPHS = 64

NC = 2
NS = 16
LW = 128

E_PAD = 163840
CH = 128
PER_TILE = E_PAD // NS
N_CHUNKS = PER_TILE // CH
ACC_ROWS = 10112
DUMMY_DST = N + 48

BM = 1000
MB = N // BM


@functools.lru_cache(maxsize=None)
def _make_agg(n_slices):
  mesh = plsc.VectorSubcoreMesh(
      core_axis_name="c", subcore_axis_name="s", num_cores=NC, num_subcores=NS)

  @functools.partial(
      pl.kernel,
      out_type=jax.ShapeDtypeStruct((n_slices, N, LW), jnp.float32),
      mesh=mesh,
      scratch_types=[
          pltpu.VMEM((CH,), jnp.int32),
          pltpu.VMEM((CH,), jnp.int32),
          pltpu.VMEM((CH,), jnp.int32),
          pltpu.VMEM((CH,), jnp.int32),
          pltpu.VMEM((CH, LW), jnp.float32),
          pltpu.VMEM((CH, LW), jnp.float32),
          pltpu.VMEM_SHARED((ACC_ROWS, LW), jnp.float32),
          pltpu.SemaphoreType.DMA,
          pltpu.SemaphoreType.DMA,
          pltpu.SemaphoreType.DMA,
          pltpu.SemaphoreType.DMA,
      ],
  )
  def agg_kernel(table_hbm, src_hbm, dst_hbm, zeros_hbm, out_hbm,
                 sidx0, sidx1, didx0, didx1, rows0, rows1, acc,
                 gsem0, gsem1, ssem0, ssem1):
    c = lax.axis_index("c")
    s = lax.axis_index("s")
    zrows = ACC_ROWS // NS

    for i in range(n_slices // NC):
      sl = i * NC + c

      pltpu.sync_copy(zeros_hbm.at[pl.ds(s * zrows, zrows)],
                      acc.at[pl.ds(s * zrows, zrows)])
      plsc.subcore_barrier()

      def body(p, carry):
        base0 = s * PER_TILE + p * (2 * CH)
        base1 = base0 + CH

        @pl.when(p > 0)
        def _():
          pltpu.make_async_copy(zeros_hbm.at[pl.ds(0, CH)], rows0, ssem0).wait()
          pltpu.make_async_copy(zeros_hbm.at[pl.ds(0, CH)], rows1, ssem1).wait()

        pltpu.sync_copy(src_hbm.at[pl.ds(base0, CH)], sidx0)
        g0 = pltpu.async_copy(table_hbm.at[sl].at[sidx0], rows0, gsem0)
        pltpu.sync_copy(src_hbm.at[pl.ds(base1, CH)], sidx1)
        g1 = pltpu.async_copy(table_hbm.at[sl].at[sidx1], rows1, gsem1)

        g0.wait()
        pltpu.sync_copy(dst_hbm.at[pl.ds(base0, CH)], didx0)
        pltpu.async_copy(rows0, acc.at[didx0], ssem0, add=True)
        g1.wait()
        pltpu.sync_copy(dst_hbm.at[pl.ds(base1, CH)], didx1)
        pltpu.async_copy(rows1, acc.at[didx1], ssem1, add=True)
        return carry

      lax.fori_loop(0, N_CHUNKS // 2, body, 0)
      pltpu.make_async_copy(zeros_hbm.at[pl.ds(0, CH)], rows0, ssem0).wait()
      pltpu.make_async_copy(zeros_hbm.at[pl.ds(0, CH)], rows1, ssem1).wait()
      plsc.subcore_barrier()

      @pl.when(s == 0)
      def _():
        pltpu.sync_copy(acc.at[pl.ds(0, 640)], out_hbm.at[sl].at[pl.ds(0, 640)])

      @pl.when(s != 0)
      def _():
        base = 640 + (s - 1) * 624
        pltpu.sync_copy(acc.at[pl.ds(base, 624)],
                        out_hbm.at[sl].at[pl.ds(base, 624)])

      plsc.subcore_barrier()

  return agg_kernel


def _mm1_body(x_ref, agg_ref, w_ref, b_ref, out_ref):
  xa = x_ref[...] + jnp.concatenate([agg_ref[0], agg_ref[1]], axis=1)
  y = jnp.dot(xa, w_ref[...], preferred_element_type=jnp.float32)
  y = y + b_ref[0]
  out_ref[...] = jnp.maximum(y, 0.0)[None]


def _mm1(x, agg1, W1, b1r):
  return pl.pallas_call(
      _mm1_body,
      grid=(MB, D_H // LW),
      in_specs=[
          pl.BlockSpec((BM, D_IN), lambda m, j: (m, 0)),
          pl.BlockSpec((2, BM, LW), lambda m, j: (0, m, 0)),
          pl.BlockSpec((D_IN, LW), lambda m, j: (0, j)),
          pl.BlockSpec((1, 1, LW), lambda m, j: (j, 0, 0)),
      ],
      out_specs=pl.BlockSpec((1, BM, LW), lambda m, j: (j, m, 0)),
      out_shape=jax.ShapeDtypeStruct((D_H // LW, N, LW), jnp.float32),
  )(x, agg1, W1, b1r)


def _mm2_body(h1_ref, agg_ref, w_ref, b_ref, batch_ref, wfc_ref, bfc_ref,
              out_ref, s_acc, c_acc):
  m = pl.program_id(0)

  @pl.when(m == 0)
  def _():
    s_acc[...] = jnp.zeros_like(s_acc)
    c_acc[...] = jnp.zeros_like(c_acc)

  hk = jnp.concatenate([h1_ref[0], h1_ref[1], h1_ref[2], h1_ref[3]], axis=1)
  ak = jnp.concatenate([agg_ref[0], agg_ref[1], agg_ref[2], agg_ref[3]],
                       axis=1)
  y = jnp.dot(hk + ak, w_ref[...], preferred_element_type=jnp.float32)
  h2 = jnp.maximum(y + b_ref[...], 0.0)

  b = batch_ref[0, 0, :]
  gids = lax.broadcasted_iota(jnp.int32, (N_GRAPHS, BM), 0)
  oh = jnp.where(b[None, :] == gids, 1.0, 0.0).astype(jnp.float32)
  s_acc[...] += jnp.dot(oh, h2, preferred_element_type=jnp.float32)
  c_acc[...] += jnp.dot(oh, jnp.ones((BM, LW), jnp.float32),
                        preferred_element_type=jnp.float32)

  @pl.when(m == MB - 1)
  def _():
    cnt = jnp.maximum(c_acc[:, 0:1], 1.0)
    pooled = s_acc[...] / cnt
    logits = jnp.dot(pooled, wfc_ref[...], preferred_element_type=jnp.float32)
    logits = logits + bfc_ref[...]
    mx = jnp.max(logits, axis=1, keepdims=True)
    z = logits - mx
    lse = jnp.log(jnp.sum(jnp.exp(z), axis=1, keepdims=True))
    out_ref[...] = z - lse


def _mm2(h1, agg2, W2, b2r, batch3, Wfc, bfcr):
  return pl.pallas_call(
      _mm2_body,
      grid=(MB,),
      in_specs=[
          pl.BlockSpec((4, BM, LW), lambda m: (0, m, 0)),
          pl.BlockSpec((4, BM, LW), lambda m: (0, m, 0)),
          pl.BlockSpec((D_H, D_H), lambda m: (0, 0)),
          pl.BlockSpec((1, D_H), lambda m: (0, 0)),
          pl.BlockSpec((1, 1, BM), lambda m: (m, 0, 0)),
          pl.BlockSpec((D_H, N_CLASSES), lambda m: (0, 0)),
          pl.BlockSpec((1, N_CLASSES), lambda m: (0, 0)),
      ],
      out_specs=pl.BlockSpec((N_GRAPHS, N_CLASSES), lambda m: (0, 0)),
      out_shape=jax.ShapeDtypeStruct((N_GRAPHS, N_CLASSES), jnp.float32),
      scratch_shapes=[
          pltpu.VMEM((N_GRAPHS, D_H), jnp.float32),
          pltpu.VMEM((N_GRAPHS, LW), jnp.float32),
      ],
  )(h1, agg2, W2, b2r, batch3, Wfc, bfcr)


def kernel(x, edge_index, batch, W1, b1, W2, b2, Wfc, bfc):
  src = edge_index[0]
  dst = edge_index[1]
  pad = E_PAD - E
  src_p = jnp.concatenate([src, jnp.zeros((pad,), jnp.int32)])
  dst_p = jnp.concatenate([dst, jnp.full((pad,), DUMMY_DST, jnp.int32)])
  zeros_acc = jnp.zeros((ACC_ROWS, LW), jnp.float32)

  x_sl = x.reshape(N, D_IN // LW, LW).transpose(1, 0, 2)
  agg1 = _make_agg(2)(x_sl, src_p, dst_p, zeros_acc)
  h1 = _mm1(x, agg1, W1, b1.reshape(D_H // LW, 1, LW))
  agg2 = _make_agg(4)(h1, src_p, dst_p, zeros_acc)
  return _mm2(h1, agg2, W2, b2.reshape(1, D_H), batch.reshape(MB, 1, BM),
              Wfc, bfc.reshape(1, N_CLASSES))


# speedup vs baseline: 2.9490x; 1.0992x over previous
import functools

import jax
import jax.numpy as jnp
from jax import lax
from jax.experimental import pallas as pl
from jax.experimental.pallas import tpu as pltpu
from jax.experimental.pallas import tpu_sc as plsc

N = 10000
E = 160000
D_IN = 256
D_H = 512
N_CLASSES = 64
N_GRAPHS = 64

NC = 2
NS = 16
LW = 128

E_PAD = 163840
CH = 128
PER_TILE = E_PAD // NS
N_CHUNKS = PER_TILE // CH
ACC_ROWS = 10112
DUMMY_DST = N + 48

BM = 1000
MB = N // BM


@functools.lru_cache(maxsize=None)
def _make_agg(n_slices):
  mesh = plsc.VectorSubcoreMesh(
      core_axis_name="c", subcore_axis_name="s", num_cores=NC, num_subcores=NS)

  @functools.partial(
      pl.kernel,
      out_type=jax.ShapeDtypeStruct((n_slices, N, LW), jnp.float32),
      mesh=mesh,
      scratch_types=[
          pltpu.VMEM((N_CHUNKS, CH), jnp.int32),
          pltpu.VMEM((CH,), jnp.int32),
          pltpu.VMEM((CH,), jnp.int32),
          pltpu.VMEM((CH, LW), jnp.float32),
          pltpu.VMEM((CH, LW), jnp.float32),
          pltpu.VMEM_SHARED((ACC_ROWS, LW), jnp.float32),
          pltpu.SemaphoreType.DMA,
          pltpu.SemaphoreType.DMA,
          pltpu.SemaphoreType.DMA,
          pltpu.SemaphoreType.DMA,
          pltpu.SemaphoreType.DMA,
          pltpu.SemaphoreType.DMA,
      ],
  )
  def agg_kernel(table_hbm, src_hbm, dst_hbm, zeros_hbm, out_hbm,
                 dstall, sidx0, sidx1, rows0, rows1, acc,
                 isem0, isem1, gsem0, gsem1, ssem0, ssem1):
    c = lax.axis_index("c")
    s = lax.axis_index("s")
    zrows = ACC_ROWS // NS
    sidx = (sidx0, sidx1)
    rows = (rows0, rows1)
    isem = (isem0, isem1)
    gsem = (gsem0, gsem1)
    ssem = (ssem0, ssem1)
    NB = 2
    NROUND = N_CHUNKS // NB

    pltpu.sync_copy(dst_hbm.at[s], dstall)

    for i in range(n_slices // NC):
      sl = i * NC + c

      pltpu.sync_copy(zeros_hbm.at[pl.ds(s * zrows, zrows)],
                      acc.at[pl.ds(s * zrows, zrows)])

      for b in range(NB):
        pltpu.async_copy(src_hbm.at[s].at[b], sidx[b], isem[b])
      plsc.subcore_barrier()

      def body(kk, carry):
        gs = []
        for b in range(NB):
          pltpu.make_async_copy(src_hbm.at[s].at[0], sidx[b], isem[b]).wait()

          @pl.when(kk > 0)
          def _(b=b):
            pltpu.make_async_copy(zeros_hbm.at[pl.ds(0, CH)], rows[b],
                                  ssem[b]).wait()

          gs.append(pltpu.async_copy(table_hbm.at[sl].at[sidx[b]],
                                     rows[b], gsem[b]))
        for b in range(NB):
          k = kk * NB + b
          gs[b].wait()

          @pl.when(kk < NROUND - 1)
          def _(b=b, k=k):
            pltpu.async_copy(src_hbm.at[s].at[k + NB], sidx[b], isem[b])

          pltpu.async_copy(rows[b], acc.at[dstall.at[k]], ssem[b], add=True)
        return carry

      lax.fori_loop(0, NROUND, body, 0)
      for b in range(NB):
        pltpu.make_async_copy(zeros_hbm.at[pl.ds(0, CH)], rows[b],
                              ssem[b]).wait()
      plsc.subcore_barrier()

      @pl.when(s == 0)
      def _():
        pltpu.sync_copy(acc.at[pl.ds(0, 640)], out_hbm.at[sl].at[pl.ds(0, 640)])

      @pl.when(s != 0)
      def _():
        base = 640 + (s - 1) * 624
        pltpu.sync_copy(acc.at[pl.ds(base, 624)],
                        out_hbm.at[sl].at[pl.ds(base, 624)])

      plsc.subcore_barrier()

  return agg_kernel


def _mm1_body(x_ref, agg_ref, w_ref, b_ref, out_ref):
  xa = x_ref[...] + jnp.concatenate([agg_ref[0], agg_ref[1]], axis=1)
  y = jnp.dot(xa, w_ref[...], preferred_element_type=jnp.float32)
  y = y + b_ref[0]
  out_ref[...] = jnp.maximum(y, 0.0)[None]


def _mm1(x, agg1, W1, b1r):
  return pl.pallas_call(
      _mm1_body,
      grid=(MB, D_H // LW),
      in_specs=[
          pl.BlockSpec((BM, D_IN), lambda m, j: (m, 0)),
          pl.BlockSpec((2, BM, LW), lambda m, j: (0, m, 0)),
          pl.BlockSpec((D_IN, LW), lambda m, j: (0, j)),
          pl.BlockSpec((1, 1, LW), lambda m, j: (j, 0, 0)),
      ],
      out_specs=pl.BlockSpec((1, BM, LW), lambda m, j: (j, m, 0)),
      out_shape=jax.ShapeDtypeStruct((D_H // LW, N, LW), jnp.float32),
  )(x, agg1, W1, b1r)


def _mm2_body(h1_ref, agg_ref, w_ref, b_ref, batch_ref, wfc_ref, bfc_ref,
              out_ref, s_acc, c_acc):
  m = pl.program_id(0)

  @pl.when(m == 0)
  def _():
    s_acc[...] = jnp.zeros_like(s_acc)
    c_acc[...] = jnp.zeros_like(c_acc)

  hk = jnp.concatenate([h1_ref[0], h1_ref[1], h1_ref[2], h1_ref[3]], axis=1)
  ak = jnp.concatenate([agg_ref[0], agg_ref[1], agg_ref[2], agg_ref[3]],
                       axis=1)
  y = jnp.dot(hk + ak, w_ref[...], preferred_element_type=jnp.float32)
  h2 = jnp.maximum(y + b_ref[...], 0.0)

  b = batch_ref[0, 0, :]
  gids = lax.broadcasted_iota(jnp.int32, (N_GRAPHS, BM), 0)
  oh = jnp.where(b[None, :] == gids, 1.0, 0.0).astype(jnp.float32)
  s_acc[...] += jnp.dot(oh, h2, preferred_element_type=jnp.float32)
  c_acc[...] += jnp.dot(oh, jnp.ones((BM, LW), jnp.float32),
                        preferred_element_type=jnp.float32)

  @pl.when(m == MB - 1)
  def _():
    cnt = jnp.maximum(c_acc[:, 0:1], 1.0)
    pooled = s_acc[...] / cnt
    logits = jnp.dot(pooled, wfc_ref[...], preferred_element_type=jnp.float32)
    logits = logits + bfc_ref[...]
    mx = jnp.max(logits, axis=1, keepdims=True)
    z = logits - mx
    lse = jnp.log(jnp.sum(jnp.exp(z), axis=1, keepdims=True))
    out_ref[...] = z - lse


def _mm2(h1, agg2, W2, b2r, batch3, Wfc, bfcr):
  return pl.pallas_call(
      _mm2_body,
      grid=(MB,),
      in_specs=[
          pl.BlockSpec((4, BM, LW), lambda m: (0, m, 0)),
          pl.BlockSpec((4, BM, LW), lambda m: (0, m, 0)),
          pl.BlockSpec((D_H, D_H), lambda m: (0, 0)),
          pl.BlockSpec((1, D_H), lambda m: (0, 0)),
          pl.BlockSpec((1, 1, BM), lambda m: (m, 0, 0)),
          pl.BlockSpec((D_H, N_CLASSES), lambda m: (0, 0)),
          pl.BlockSpec((1, N_CLASSES), lambda m: (0, 0)),
      ],
      out_specs=pl.BlockSpec((N_GRAPHS, N_CLASSES), lambda m: (0, 0)),
      out_shape=jax.ShapeDtypeStruct((N_GRAPHS, N_CLASSES), jnp.float32),
      scratch_shapes=[
          pltpu.VMEM((N_GRAPHS, D_H), jnp.float32),
          pltpu.VMEM((N_GRAPHS, LW), jnp.float32),
      ],
  )(h1, agg2, W2, b2r, batch3, Wfc, bfcr)


def kernel(x, edge_index, batch, W1, b1, W2, b2, Wfc, bfc):
  src = edge_index[0]
  dst = edge_index[1]
  pad = E_PAD - E
  src_p = jnp.concatenate([src, jnp.zeros((pad,), jnp.int32)])
  dst_p = jnp.concatenate([dst, jnp.full((pad,), DUMMY_DST, jnp.int32)])
  src_p = src_p.reshape(NS, N_CHUNKS, CH)
  dst_p = dst_p.reshape(NS, N_CHUNKS, CH)
  zeros_acc = jnp.zeros((ACC_ROWS, LW), jnp.float32)

  x_sl = x.reshape(N, D_IN // LW, LW).transpose(1, 0, 2)
  agg1 = _make_agg(2)(x_sl, src_p, dst_p, zeros_acc)
  h1 = _mm1(x, agg1, W1, b1.reshape(D_H // LW, 1, LW))
  agg2 = _make_agg(4)(h1, src_p, dst_p, zeros_acc)
  return _mm2(h1, agg2, W2, b2.reshape(1, D_H), batch.reshape(MB, 1, BM),
              Wfc, bfc.reshape(1, N_CLASSES))
